# baseline (speedup 1.0000x reference)
import jax
import jax.numpy as jnp
from jax import lax
from jax.experimental import pallas as pl
from jax.experimental.pallas import tpu as pltpu
from jax.experimental.pallas import tpu_sc as plsc

D = 128
B = 16384
NC = 2
NS = 16
L = 16
NW = NC * NS
RPW = B // NW
C = 64
NCHUNK = RPW // C
NBUF = 3


def _rsqrt_newton(x):
    bits = plsc.bitcast(x, jnp.int32)
    y = plsc.bitcast(jnp.int32(0x5F3759DF) - (bits >> 1), jnp.float32)
    for _ in range(3):
        y = y * (1.5 - 0.5 * x * y * y)
    return y


def _body(head_hbm, rel_hbm, tail_hbm, ent_hbm, relemb_hbm, out_hbm,
          idx_ht, idx_r, outv, *scratch):
    bufs = tuple((scratch[2 * i], scratch[2 * i + 1],
                  scratch[2 * NBUF + 1 + i]) for i in range(NBUF))
    isem = scratch[2 * NBUF]

    wid = lax.axis_index("s") * NC + lax.axis_index("c")
    base = wid * RPW

    def stage(c, sem):
        return (
            pltpu.async_copy(head_hbm.at[pl.ds(base + c * C, C)],
                             idx_ht.at[pl.ds(c * 2 * C, C)], sem),
            pltpu.async_copy(tail_hbm.at[pl.ds(base + c * C, C)],
                             idx_ht.at[pl.ds(c * 2 * C + C, C)], sem),
            pltpu.async_copy(rel_hbm.at[pl.ds(base + c * C, C)],
                             idx_r.at[pl.ds(c * C, C)], sem),
        )

    d_a = stage(0, isem)
    sem_b = bufs[NBUF - 1][2]
    d_b = [dsc for c in range(1, NCHUNK) for dsc in stage(c, sem_b)]

    def fire(c):
        htb, rb, sem = bufs[c % NBUF]
        return (
            pltpu.async_copy(ent_hbm.at[idx_ht.at[pl.ds(c * 2 * C, 2 * C)]],
                             htb, sem),
            pltpu.async_copy(relemb_hbm.at[idx_r.at[pl.ds(c * C, C)]],
                             rb, sem),
        )

    for dsc in d_a:
        dsc.wait()
    descs = [fire(0)]
    for dsc in d_b:
        dsc.wait()
    descs += [fire(c) for c in range(1, min(NBUF, NCHUNK))]
    odescs = []
    lane = lax.broadcasted_iota(jnp.int32, (L,), 0)
    for c in range(NCHUNK):
        for dsc in descs[c % NBUF]:
            dsc.wait()
        htb, rb, _ = bufs[c % NBUF]
        for g in range(C // L):
            row = lane + g * L
            row_t = row + C

            def dim_step(carry):
                acc, offs = carry
                vh = plsc.load_gather(htb, [row, offs])
                vr = plsc.load_gather(rb, [row, offs])
                vt = plsc.load_gather(htb, [row_t, offs])
                dif = (vh + vr) - vt
                return acc + dif * dif, (offs + 1) & (D - 1)

            def dim_step4(_, carry):
                for _u in range(4):
                    carry = dim_step(carry)
                return carry

            acc, _ = lax.fori_loop(0, D // 4, dim_step4,
                                   (jnp.zeros((L,), jnp.float32), lane))
            acc_s = jnp.maximum(acc, jnp.float32(1e-12))
            outv[pl.ds(c * C + g * L, L)] = acc * _rsqrt_newton(acc_s)
        if c + NBUF < NCHUNK:
            descs[c % NBUF] = fire(c + NBUF)
        odescs.append(pltpu.async_copy(outv.at[pl.ds(c * C, C)],
                                       out_hbm.at[pl.ds(base + c * C, C)],
                                       isem))

    for dsc in odescs:
        dsc.wait()


@jax.jit
def _transe_sc(head, relation, tail, entity_embeddings, relation_embeddings):
    mesh = plsc.VectorSubcoreMesh(core_axis_name="c", subcore_axis_name="s",
                                  num_cores=NC, num_subcores=NS)
    scratch = (
        [pltpu.VMEM((2 * RPW,), jnp.int32)]
        + [pltpu.VMEM((RPW,), jnp.int32)]
        + [pltpu.VMEM((RPW,), jnp.float32)]
        + [pltpu.VMEM((2 * C, D), jnp.float32),
           pltpu.VMEM((C, D), jnp.float32)] * NBUF
        + [pltpu.SemaphoreType.DMA] * (1 + NBUF)
    )
    return pl.kernel(
        _body,
        out_type=jax.ShapeDtypeStruct((B,), jnp.float32),
        mesh=mesh,
        compiler_params=pltpu.CompilerParams(needs_layout_passes=False),
        scratch_types=scratch,
    )(head, relation, tail, entity_embeddings, relation_embeddings)


def kernel(head, relation, tail, entity_embeddings, relation_embeddings):
    return _transe_sc(head, relation, tail, entity_embeddings,
                      relation_embeddings)
